# baseline (speedup 1.0000x reference)
import functools

import jax
import jax.numpy as jnp
from jax import lax
from jax.experimental import pallas as pl
from jax.experimental.pallas import tpu as pltpu
from jax.experimental.pallas import tpu_sc as plsc

f32 = jnp.float32
i32 = jnp.int32

_N = 10000
_E = 320000
_L = 128
_EMB = 16
_C = 500
_CP = 512
_KS = 8
_KB = 4
_ITERS = 2
_SE = 2 * _C * _KS
_NB = 2000
_EB = 2000
_SEB = 2000
_CHUNK = 40
_NW = 32
_EPW = _E // _NW
_NCH = _EPW // _CHUNK
_GBUF_G = 10
_GBUF_S = 5


def _ln(x):
    m = x.mean(-1, keepdims=True)
    v = ((x - m) ** 2).mean(-1, keepdims=True)
    return (x - m) * lax.rsqrt(v + 1e-5)


def _dot(a, b):
    return jnp.dot(a, b, preferred_element_type=f32)


def _dotT(a, b):
    return lax.dot_general(a, b, (((0,), (0,)), ((), ())),
                           preferred_element_type=f32)


def _iota_r(n):
    return lax.broadcasted_iota(i32, (1, n), 1)


def _iota_c(n):
    return lax.broadcasted_iota(i32, (n, 1), 0)


def _k_means_body(emb_b, cl_b, o_meansT, acc, cnt):
    step = pl.program_id(0)

    @pl.when(step == 0)
    def _():
        acc[...] = jnp.zeros_like(acc)
        cnt[...] = jnp.zeros_like(cnt)

    hc = (cl_b[...] == _iota_r(_CP)).astype(f32)
    acc[...] += _dotT(emb_b[...], hc)
    cnt[...] += jnp.sum(hc, axis=0, keepdims=True)

    @pl.when(step == pl.num_programs(0) - 1)
    def _():
        mT = acc[...] / jnp.maximum(cnt[...], 1.0)
        nrm = jnp.sqrt(jnp.sum(mT * mT, axis=0, keepdims=True))
        o_meansT[...] = mT / (nrm + 1e-12)


def _tc_means(emb, clusters_col):
    grid = _N // _NB
    return pl.pallas_call(
        _k_means_body,
        grid=(grid,),
        in_specs=[
            pl.BlockSpec((_NB, _EMB), lambda i: (i, 0)),
            pl.BlockSpec((_NB, 1), lambda i: (i, 0)),
        ],
        out_specs=pl.BlockSpec((_EMB, _CP), lambda i: (0, 0)),
        out_shape=jax.ShapeDtypeStruct((_EMB, _CP), f32),
        scratch_shapes=[pltpu.VMEM((_EMB, _CP), f32), pltpu.VMEM((1, _CP), f32)],
    )(emb, clusters_col)


def _k_super_body(means_r, meansT_r, wb_r, o_idx, o_sew):
    m = means_r[...]
    mT = meansT_r[...]
    r2 = jnp.sum(m * m, axis=1, keepdims=True)
    c2 = jnp.sum(mT * mT, axis=0, keepdims=True)
    d2 = jnp.maximum(r2 + c2 - 2.0 * _dot(m, mT), 0.0)
    ic, ir = _iota_c(_CP), _iota_r(_CP)
    bad = (ic == ir) | (ir >= _C)
    x = -(d2 + jnp.where(bad, 1e9, 0.0))
    idxs, vals = [], []
    for _ in range(_KS):
        best = jnp.max(x, axis=1, keepdims=True)
        am = jnp.min(jnp.where(x == best, ir, _CP), axis=1, keepdims=True)
        idxs.append(am)
        vals.append(best)
        x = jnp.where(ir == am, -jnp.inf, x)
    o_idx[...] = jnp.concatenate(idxs, axis=1)
    negd = jnp.concatenate(vals, axis=1)
    w = wb_r[0, 0]
    b = wb_r[0, 1]
    o_sew[...] = jax.nn.sigmoid(negd * w + b)


def _tc_super(means, meansT, wb):
    return pl.pallas_call(
        _k_super_body,
        in_specs=[
            pl.BlockSpec((_CP, _EMB), lambda: (0, 0)),
            pl.BlockSpec((_EMB, _CP), lambda: (0, 0)),
            pl.BlockSpec((1, 2), lambda: (0, 0)),
        ],
        out_specs=[
            pl.BlockSpec((_CP, _KS), lambda: (0, 0)),
            pl.BlockSpec((_CP, _KS), lambda: (0, 0)),
        ],
        out_shape=[
            jax.ShapeDtypeStruct((_CP, _KS), i32),
            jax.ShapeDtypeStruct((_CP, _KS), f32),
        ],
    )(means, meansT, wb)


def _k_bi_body(emb_b, meansT_r, w_r, o_mw, o_dinv, accd):
    step = pl.program_id(0)

    @pl.when(step == 0)
    def _():
        accd[...] = jnp.zeros_like(accd)

    e = emb_b[...]
    mT = meansT_r[...]
    e2 = jnp.sum(e * e, axis=1, keepdims=True)
    m2 = jnp.sum(mT * mT, axis=0, keepdims=True)
    d2 = jnp.maximum(e2 + m2 - 2.0 * _dot(e, mT), 0.0)
    ir = _iota_r(_CP)
    x = -(d2 + jnp.where(ir >= _C, 1e9, 0.0))
    w = w_r[0, 0]
    mw = jnp.zeros_like(d2)
    for _ in range(_KB):
        best = jnp.max(x, axis=1, keepdims=True)
        am = jnp.min(jnp.where(x == best, ir, _CP), axis=1, keepdims=True)
        wk = jnp.exp(best * w)
        mw = mw + jnp.where(ir == am, wk, 0.0)
        x = jnp.where(ir == am, -jnp.inf, x)
    o_mw[...] = mw
    accd[...] += jnp.sum(mw, axis=0, keepdims=True)
    o_dinv[...] = 1.0 / jnp.maximum(accd[...], 1e-12)


def _tc_bi(emb, meansT, w):
    grid = _N // _NB
    return pl.pallas_call(
        _k_bi_body,
        grid=(grid,),
        in_specs=[
            pl.BlockSpec((_NB, _EMB), lambda i: (i, 0)),
            pl.BlockSpec((_EMB, _CP), lambda i: (0, 0)),
            pl.BlockSpec((1, 1), lambda i: (0, 0)),
        ],
        out_specs=[
            pl.BlockSpec((_NB, _CP), lambda i: (i, 0)),
            pl.BlockSpec((1, _CP), lambda i: (0, 0)),
        ],
        out_shape=[
            jax.ShapeDtypeStruct((_N, _CP), f32),
            jax.ShapeDtypeStruct((1, _CP), f32),
        ],
        scratch_shapes=[pltpu.VMEM((1, _CP), f32)],
    )(emb, meansT, w)


def _k_sninit_body(mw_b, nodes_b, means_r, dinvT_r, w1_r, b1_r, w2_r, b2_r,
                   w1a_r, w1b_r, o_s, o_a, o_b, acc):
    step = pl.program_id(0)

    @pl.when(step == 0)
    def _():
        acc[...] = jnp.zeros_like(acc)

    nb = nodes_b[...]
    acc[...] += _dotT(mw_b[...], nb)
    o_a[...] = _dot(nb, w1a_r[...])
    o_b[...] = _dot(nb, w1b_r[...])

    @pl.when(step == pl.num_programs(0) - 1)
    def _():
        snr = acc[...] * dinvT_r[...]
        h = _ln(jax.nn.relu(_dot(snr, w1_r[...]) + b1_r[...]))
        o = _ln(jax.nn.relu(_dot(h, w2_r[...]) + b2_r[...]))
        o_s[...] = jnp.concatenate([means_r[...], o], axis=1)


def _tc_sn_init(mw, nodes, means, dinvT, w1, b1, w2, b2, w1a, w1b):
    grid = _N // _NB
    return pl.pallas_call(
        _k_sninit_body,
        grid=(grid,),
        in_specs=[
            pl.BlockSpec((_NB, _CP), lambda i: (i, 0)),
            pl.BlockSpec((_NB, _L), lambda i: (i, 0)),
            pl.BlockSpec((_CP, _EMB), lambda i: (0, 0)),
            pl.BlockSpec((_CP, 1), lambda i: (0, 0)),
            pl.BlockSpec((_L, _L), lambda i: (0, 0)),
            pl.BlockSpec((1, _L), lambda i: (0, 0)),
            pl.BlockSpec((_L, _L - _EMB), lambda i: (0, 0)),
            pl.BlockSpec((1, _L - _EMB), lambda i: (0, 0)),
            pl.BlockSpec((_L, _L), lambda i: (0, 0)),
            pl.BlockSpec((_L, _L), lambda i: (0, 0)),
        ],
        out_specs=[
            pl.BlockSpec((_CP, _L), lambda i: (0, 0)),
            pl.BlockSpec((_NB, _L), lambda i: (i, 0)),
            pl.BlockSpec((_NB, _L), lambda i: (i, 0)),
        ],
        out_shape=[
            jax.ShapeDtypeStruct((_CP, _L), f32),
            jax.ShapeDtypeStruct((_N, _L), f32),
            jax.ShapeDtypeStruct((_N, _L), f32),
        ],
        scratch_shapes=[pltpu.VMEM((_CP, _L), f32)],
    )(mw, nodes, means, dinvT, w1, b1, w2, b2, w1a, w1b)


def _k_aggn2s_body(mw_b, nodes_b, dinvT_r, o_agg, acc):
    step = pl.program_id(0)

    @pl.when(step == 0)
    def _():
        acc[...] = jnp.zeros_like(acc)

    acc[...] += _dotT(mw_b[...], nodes_b[...])

    @pl.when(step == pl.num_programs(0) - 1)
    def _():
        o_agg[...] = acc[...] * dinvT_r[...]


def _tc_aggn2s(mw, nodes, dinvT):
    grid = _N // _NB
    return pl.pallas_call(
        _k_aggn2s_body,
        grid=(grid,),
        in_specs=[
            pl.BlockSpec((_NB, _CP), lambda i: (i, 0)),
            pl.BlockSpec((_NB, _L), lambda i: (i, 0)),
            pl.BlockSpec((_CP, 1), lambda i: (0, 0)),
        ],
        out_specs=pl.BlockSpec((_CP, _L), lambda i: (0, 0)),
        out_shape=jax.ShapeDtypeStruct((_CP, _L), f32),
        scratch_shapes=[pltpu.VMEM((_CP, _L), f32)],
    )(mw, nodes, dinvT)


def _k_seinit_body(sg0_b, sg1_b, s_r, wa_r, wb_r, b1_r, w2_r, b2_r, o_se):
    ir = _iota_r(_CP)
    s = s_r[...]
    h0 = (sg0_b[...] == ir).astype(f32)
    h1 = (sg1_b[...] == ir).astype(f32)
    h = _ln(jax.nn.relu(_dot(_dot(h0, s), wa_r[...])
                        + _dot(_dot(h1, s), wb_r[...]) + b1_r[...]))
    o_se[...] = _ln(jax.nn.relu(_dot(h, w2_r[...]) + b2_r[...]))


def _tc_se_init(sg0c, sg1c, s, wa, wb, b1, w2, b2):
    grid = _SE // _SEB
    return pl.pallas_call(
        _k_seinit_body,
        grid=(grid,),
        in_specs=[
            pl.BlockSpec((_SEB, 1), lambda i: (i, 0)),
            pl.BlockSpec((_SEB, 1), lambda i: (i, 0)),
            pl.BlockSpec((_CP, _L), lambda i: (0, 0)),
            pl.BlockSpec((_L, _L), lambda i: (0, 0)),
            pl.BlockSpec((_L, _L), lambda i: (0, 0)),
            pl.BlockSpec((1, _L), lambda i: (0, 0)),
            pl.BlockSpec((_L, _L), lambda i: (0, 0)),
            pl.BlockSpec((1, _L), lambda i: (0, 0)),
        ],
        out_specs=pl.BlockSpec((_SEB, _L), lambda i: (i, 0)),
        out_shape=jax.ShapeDtypeStruct((_SE, _L), f32),
    )(sg0c, sg1c, s, wa, wb, b1, w2, b2)


def _k_seupd_body(sg0_b, sg1_b, sg1r_b, sew_b, se_b, s_r,
                  wa_r, wb_r, wc_r, b1_r, w2_r, b2_r, o_se, o_agg):
    step = pl.program_id(0)

    @pl.when(step == 0)
    def _():
        o_agg[...] = jnp.zeros_like(o_agg)

    ir = _iota_r(_CP)
    s = s_r[...]
    se = se_b[...]
    h0 = (sg0_b[...] == ir).astype(f32)
    h1 = (sg1_b[...] == ir).astype(f32)
    h = _ln(jax.nn.relu(_dot(_dot(h0, s), wa_r[...])
                        + _dot(_dot(h1, s), wb_r[...])
                        + _dot(se, wc_r[...]) + b1_r[...]))
    se_new = se + _ln(jax.nn.relu(_dot(h, w2_r[...]) + b2_r[...]))
    o_se[...] = se_new
    h1t = (_iota_c(_CP) == sg1r_b[0]).astype(f32)
    o_agg[...] += _dot(h1t, se_new * sew_b[...])


def _tc_se_update(sg0c, sg1c, sg1r3, sewc, se, s, wa, wb, wc, b1, w2, b2):
    grid = _SE // _SEB
    return pl.pallas_call(
        _k_seupd_body,
        grid=(grid,),
        in_specs=[
            pl.BlockSpec((_SEB, 1), lambda i: (i, 0)),
            pl.BlockSpec((_SEB, 1), lambda i: (i, 0)),
            pl.BlockSpec((1, 1, _SEB), lambda i: (i, 0, 0)),
            pl.BlockSpec((_SEB, 1), lambda i: (i, 0)),
            pl.BlockSpec((_SEB, _L), lambda i: (i, 0)),
            pl.BlockSpec((_CP, _L), lambda i: (0, 0)),
            pl.BlockSpec((_L, _L), lambda i: (0, 0)),
            pl.BlockSpec((_L, _L), lambda i: (0, 0)),
            pl.BlockSpec((_L, _L), lambda i: (0, 0)),
            pl.BlockSpec((1, _L), lambda i: (0, 0)),
            pl.BlockSpec((_L, _L), lambda i: (0, 0)),
            pl.BlockSpec((1, _L), lambda i: (0, 0)),
        ],
        out_specs=[
            pl.BlockSpec((_SEB, _L), lambda i: (i, 0)),
            pl.BlockSpec((_CP, _L), lambda i: (0, 0)),
        ],
        out_shape=[
            jax.ShapeDtypeStruct((_SE, _L), f32),
            jax.ShapeDtypeStruct((_CP, _L), f32),
        ],
    )(sg0c, sg1c, sg1r3, sewc, se, s, wa, wb, wc, b1, w2, b2)


def _k_snupd_body(s_r, aggse_r, aggn2s_r, dinvT_r,
                  wa_r, wb_r, wc_r, b1_r, w2_r, b2_r, o_s, o_ssc):
    s = s_r[...]
    h = _ln(jax.nn.relu(_dot(s, wa_r[...]) + _dot(aggse_r[...], wb_r[...])
                        + _dot(aggn2s_r[...], wc_r[...]) + b1_r[...]))
    s_new = s + _ln(jax.nn.relu(_dot(h, w2_r[...]) + b2_r[...]))
    o_s[...] = s_new
    o_ssc[...] = s_new * dinvT_r[...]


def _tc_sn_update(s, aggse, aggn2s, dinvT, wa, wb, wc, b1, w2, b2):
    specs = [
        pl.BlockSpec((_CP, _L), lambda: (0, 0)),
        pl.BlockSpec((_CP, _L), lambda: (0, 0)),
        pl.BlockSpec((_CP, _L), lambda: (0, 0)),
        pl.BlockSpec((_CP, 1), lambda: (0, 0)),
        pl.BlockSpec((_L, _L), lambda: (0, 0)),
        pl.BlockSpec((_L, _L), lambda: (0, 0)),
        pl.BlockSpec((_L, _L), lambda: (0, 0)),
        pl.BlockSpec((1, _L), lambda: (0, 0)),
        pl.BlockSpec((_L, _L), lambda: (0, 0)),
        pl.BlockSpec((1, _L), lambda: (0, 0)),
    ]
    return pl.pallas_call(
        _k_snupd_body,
        in_specs=specs,
        out_specs=[
            pl.BlockSpec((_CP, _L), lambda: (0, 0)),
            pl.BlockSpec((_CP, _L), lambda: (0, 0)),
        ],
        out_shape=[
            jax.ShapeDtypeStruct((_CP, _L), f32),
            jax.ShapeDtypeStruct((_CP, _L), f32),
        ],
    )(s, aggse, aggn2s, dinvT, wa, wb, wc, b1, w2, b2)


def _k_edge_body(g_b, e_b, wc_r, b1_r, w2_r, b2_r, o_e):
    e = e_b[...]
    h = _ln(jax.nn.relu(g_b[...] + _dot(e, wc_r[...]) + b1_r[...]))
    o_e[...] = e + _ln(jax.nn.relu(_dot(h, w2_r[...]) + b2_r[...]))


def _tc_edge(g, e, wc, b1, w2, b2):
    grid = _E // _EB
    return pl.pallas_call(
        _k_edge_body,
        grid=(grid,),
        in_specs=[
            pl.BlockSpec((_EB, _L), lambda i: (i, 0)),
            pl.BlockSpec((_EB, _L), lambda i: (i, 0)),
            pl.BlockSpec((_L, _L), lambda i: (0, 0)),
            pl.BlockSpec((1, _L), lambda i: (0, 0)),
            pl.BlockSpec((_L, _L), lambda i: (0, 0)),
            pl.BlockSpec((1, _L), lambda i: (0, 0)),
        ],
        out_specs=pl.BlockSpec((_EB, _L), lambda i: (i, 0)),
        out_shape=jax.ShapeDtypeStruct((_E, _L), f32),
    )(g, e, wc, b1, w2, b2)


def _k_node_body(nodes_b, p0_b, p1_b, mw_b, ssc_r,
                 wa_r, wb_r, wc_r, b1_r, w2_r, b2_r, w1a_r, w1b_r,
                 o_n, o_a, o_b):
    n = nodes_b[...]
    agge = p0_b[...] + p1_b[...]
    aggs2n = _dot(mw_b[...], ssc_r[...])
    h = _ln(jax.nn.relu(_dot(n, wa_r[...]) + _dot(agge, wb_r[...])
                        + _dot(aggs2n, wc_r[...]) + b1_r[...]))
    n_new = n + _ln(jax.nn.relu(_dot(h, w2_r[...]) + b2_r[...]))
    o_n[...] = n_new
    o_a[...] = _dot(n_new, w1a_r[...])
    o_b[...] = _dot(n_new, w1b_r[...])


def _tc_node(nodes, p0, p1, mw, ssc, wa, wb, wc, b1, w2, b2, w1a, w1b):
    grid = _N // _NB
    return pl.pallas_call(
        _k_node_body,
        grid=(grid,),
        in_specs=[
            pl.BlockSpec((_NB, _L), lambda i: (i, 0)),
            pl.BlockSpec((_NB, _L), lambda i: (i, 0)),
            pl.BlockSpec((_NB, _L), lambda i: (i, 0)),
            pl.BlockSpec((_NB, _CP), lambda i: (i, 0)),
            pl.BlockSpec((_CP, _L), lambda i: (0, 0)),
            pl.BlockSpec((_L, _L), lambda i: (0, 0)),
            pl.BlockSpec((_L, _L), lambda i: (0, 0)),
            pl.BlockSpec((_L, _L), lambda i: (0, 0)),
            pl.BlockSpec((1, _L), lambda i: (0, 0)),
            pl.BlockSpec((_L, _L), lambda i: (0, 0)),
            pl.BlockSpec((1, _L), lambda i: (0, 0)),
            pl.BlockSpec((_L, _L), lambda i: (0, 0)),
            pl.BlockSpec((_L, _L), lambda i: (0, 0)),
        ],
        out_specs=[
            pl.BlockSpec((_NB, _L), lambda i: (i, 0)),
            pl.BlockSpec((_NB, _L), lambda i: (i, 0)),
            pl.BlockSpec((_NB, _L), lambda i: (i, 0)),
        ],
        out_shape=[
            jax.ShapeDtypeStruct((_N, _L), f32),
            jax.ShapeDtypeStruct((_N, _L), f32),
            jax.ShapeDtypeStruct((_N, _L), f32),
        ],
    )(nodes, p0, p1, mw, ssc, wa, wb, wc, b1, w2, b2, w1a, w1b)


def _k_out_body(nodes_b, w1_r, b1_r, w2_r, b2_r, o_b):
    h = _ln(jax.nn.relu(_dot(nodes_b[...], w1_r[...]) + b1_r[...]))
    o = _dot(h, w2_r[...]) + b2_r[...]
    nrm = jnp.sqrt(jnp.sum(o * o, axis=1, keepdims=True))
    o_b[...] = o / (nrm + 1e-12)


def _tc_out(nodes, w1, b1, w2, b2):
    grid = _N // _NB
    return pl.pallas_call(
        _k_out_body,
        grid=(grid,),
        in_specs=[
            pl.BlockSpec((_NB, _L), lambda i: (i, 0)),
            pl.BlockSpec((_L, _L), lambda i: (0, 0)),
            pl.BlockSpec((1, _L), lambda i: (0, 0)),
            pl.BlockSpec((_L, _EMB), lambda i: (0, 0)),
            pl.BlockSpec((1, _EMB), lambda i: (0, 0)),
        ],
        out_specs=pl.BlockSpec((_NB, _EMB), lambda i: (i, 0)),
        out_shape=jax.ShapeDtypeStruct((_N, _EMB), f32),
    )(nodes, w1, b1, w2, b2)


def _sc_gather(a, b, g0r, g1r):
    mesh = plsc.VectorSubcoreMesh(core_axis_name="c", subcore_axis_name="s")

    @functools.partial(
        pl.kernel, mesh=mesh,
        out_type=jax.ShapeDtypeStruct((_E, _L), f32),
        scratch_types=[
            pltpu.VMEM((_GBUF_G, _CHUNK), i32),
            pltpu.VMEM((_GBUF_G, _CHUNK), i32),
            [pltpu.VMEM((_CHUNK, _L), f32)] * _GBUF_G,
            [pltpu.VMEM((_CHUNK, _L), f32)] * _GBUF_G,
            [pltpu.SemaphoreType.DMA] * _GBUF_G,
        ],
        name="sc_gather_edges",
    )
    def k(a_h, b_h, g0_h, g1_h, out_h, i0, i1, bas, bbs, sems):
        wid = lax.axis_index("s") * 2 + lax.axis_index("c")

        def group(gg, carry):
            j0 = gg * _GBUF_G
            pltpu.sync_copy(g0_h.at[wid, gg], i0)
            pltpu.sync_copy(g1_h.at[wid, gg], i1)
            handles = []
            for p in range(_GBUF_G):
                ha = pltpu.async_copy(a_h.at[i0.at[p]], bas[p], sems[p])
                hb = pltpu.async_copy(b_h.at[i1.at[p]], bbs[p], sems[p])
                handles.append((ha, hb))
            for p in range(_GBUF_G):
                ha, hb = handles[p]
                ha.wait()
                hb.wait()
                ba, bb = bas[p], bbs[p]

                def addrow(r, c2):
                    for cc in range(_L // 16):
                        sl = pl.ds(cc * 16, 16)
                        ba[r, sl] = ba[r, sl] + bb[r, sl]
                    return c2

                lax.fori_loop(0, _CHUNK, addrow, 0, unroll=4)
                base = wid * _EPW + (j0 + p) * _CHUNK
                pltpu.sync_copy(ba, out_h.at[pl.ds(base, _CHUNK)])
            return carry

        lax.fori_loop(0, _NCH // _GBUF_G, group, 0)

    return k(a, b, g0r, g1r)


_NPAD = 10240


def _sc_scatter(vals, g1r):
    mesh = plsc.VectorSubcoreMesh(core_axis_name="c", subcore_axis_name="s")
    rows_per_sub = _NPAD // 16

    @functools.partial(
        pl.kernel, mesh=mesh,
        out_type=jax.ShapeDtypeStruct((2, _NPAD, _L), f32),
        scratch_types=[
            pltpu.VMEM((_GBUF_S, _CHUNK), i32),
            [pltpu.VMEM((_CHUNK, _L), f32)] * _GBUF_S,
            pltpu.VMEM((80, _L), f32),
            pltpu.VMEM_SHARED((_NPAD, _L), f32),
            [pltpu.SemaphoreType.DMA] * _GBUF_S,
            pltpu.SemaphoreType.DMA,
        ],
        name="sc_scatter_edges",
    )
    def k(v_h, g1_h, out_h, idx, bufs, zbuf, acc, sems, sem_s):
        cid = lax.axis_index("c")
        sid = lax.axis_index("s")
        wid = sid * 2 + cid

        def zrow(r, c2):
            for cc in range(_L // 16):
                zbuf[r, pl.ds(cc * 16, 16)] = jnp.zeros((16,), f32)
            return c2

        lax.fori_loop(0, 80, zrow, 0)
        for t in range(rows_per_sub // 80):
            pltpu.sync_copy(zbuf, acc.at[pl.ds(sid * rows_per_sub + t * 80,
                                               80)])
        plsc.subcore_barrier()

        def group(gg, carry):
            j0 = gg * _GBUF_S
            pltpu.sync_copy(g1_h.at[wid, gg], idx)
            handles = []
            for p in range(_GBUF_S):
                base = wid * _EPW + (j0 + p) * _CHUNK
                handles.append(pltpu.async_copy(
                    v_h.at[pl.ds(base, _CHUNK)], bufs[p], sems[p]))
            sc_handles = []
            for p in range(_GBUF_S):
                handles[p].wait()
                sc_handles.append(pltpu.async_copy(
                    bufs[p], acc.at[idx.at[p]], sem_s, add=True))
            for h in sc_handles:
                h.wait()
            return carry

        lax.fori_loop(0, _NCH // _GBUF_S, group, 0)
        plsc.subcore_barrier()
        pltpu.sync_copy(acc.at[pl.ds(sid * rows_per_sub, rows_per_sub)],
                        out_h.at[cid, pl.ds(sid * rows_per_sub, rows_per_sub)])

    return k(vals, g1r)


def kernel(x, embeddings, nodes, edges, graph, clusters, params):
    p = params
    ew1, eb1 = p["cell_edge"][0]
    ew2, eb2 = p["cell_edge"][1]
    ew1a, ew1b, ew1c = ew1[:_L], ew1[_L:2 * _L], ew1[2 * _L:]

    clc = clusters.reshape(_N, 1)
    meansT = _tc_means(embeddings, clc)
    means = meansT.T

    wb_super = jnp.stack([p["gc_super_w"], p["gc_super_b"]]).reshape(1, 2)
    idxs, sewk = _tc_super(means, meansT, wb_super)
    src = idxs[:_C].reshape(-1)
    dst = jnp.repeat(jnp.arange(_C, dtype=i32), _KS)
    sg0 = jnp.concatenate([src, dst])
    sg1 = jnp.concatenate([dst, src])
    sewh = sewk[:_C].reshape(-1)
    sew = jnp.concatenate([sewh, sewh])
    sg0c = sg0.reshape(_SE, 1)
    sg1c = sg1.reshape(_SE, 1)
    sg1r3 = sg1.reshape(_SE // _SEB, 1, _SEB)
    sewc = sew.reshape(_SE, 1)

    mw, dinv = _tc_bi(embeddings, meansT, p["gc_bi_w"].reshape(1, 1))
    dinvT = dinv.reshape(_CP, 1)

    (sw1, sb1), (sw2, sb2) = p["sn_enc"]
    s, a, b = _tc_sn_init(mw, nodes, means, dinvT, sw1, sb1.reshape(1, -1),
                          sw2, sb2.reshape(1, -1), ew1a, ew1b)

    (qw1, qb1), (qw2, qb2) = p["se_enc"]
    se = _tc_se_init(sg0c, sg1c, s, qw1[:_L], qw1[_L:], qb1.reshape(1, -1),
                     qw2, qb2.reshape(1, -1))

    g0r = graph[0].reshape(_NW, _NCH // _GBUF_G, _GBUF_G, _CHUNK)
    g1rg = graph[1].reshape(_NW, _NCH // _GBUF_G, _GBUF_G, _CHUNK)
    g1rs = graph[1].reshape(_NW, _NCH // _GBUF_S, _GBUF_S, _CHUNK)

    (dw1, db1), (dw2, db2) = p["cell_sedge"]
    (uw1, ub1), (uw2, ub2) = p["cell_snode"]
    (nw1, nb1), (nw2, nb2) = p["cell_node"]

    for _ in range(_ITERS):
        g = _sc_gather(a, b, g0r, g1rg)
        edges = _tc_edge(g, edges, ew1c, eb1.reshape(1, -1), ew2,
                         eb2.reshape(1, -1))
        parts = _sc_scatter(edges, g1rs)[:, :_N]
        aggn2s = _tc_aggn2s(mw, nodes, dinvT)
        se, aggse = _tc_se_update(sg0c, sg1c, sg1r3, sewc, se, s,
                                  dw1[:_L], dw1[_L:2 * _L], dw1[2 * _L:],
                                  db1.reshape(1, -1), dw2, db2.reshape(1, -1))
        s, ssc = _tc_sn_update(s, aggse, aggn2s, dinvT,
                               uw1[:_L], uw1[_L:2 * _L], uw1[2 * _L:],
                               ub1.reshape(1, -1), uw2, ub2.reshape(1, -1))
        nodes, a, b = _tc_node(nodes, parts[0], parts[1], mw, ssc,
                               nw1[:_L], nw1[_L:2 * _L], nw1[2 * _L:],
                               nb1.reshape(1, -1), nw2, nb2.reshape(1, -1),
                               ew1a, ew1b)

    (ow1, ob1), (ow2, ob2) = p["out"]
    out = _tc_out(nodes, ow1, ob1.reshape(1, -1), ow2, ob2.reshape(1, -1))
    return (out, clusters)


# speedup vs baseline: 1.0017x; 1.0017x over previous
import functools

import jax
import jax.numpy as jnp
from jax import lax
from jax.experimental import pallas as pl
from jax.experimental.pallas import tpu as pltpu
from jax.experimental.pallas import tpu_sc as plsc

f32 = jnp.float32
i32 = jnp.int32

_N = 10000
_E = 320000
_L = 128
_EMB = 16
_C = 500
_CP = 512
_KS = 8
_KB = 4
_ITERS = 2
_SE = 2 * _C * _KS
_NB = 2000
_EB = 2000
_SEB = 2000
_NW = 32
_EPW = _E // _NW
_CHG = 128
_NCHG = 78
_TAIL = _EPW - _NCHG * _CHG
_GBUF_G = 3
_CHS = 40
_NCHS = _EPW // _CHS
_GBUF_S = 5


def _ln(x):
    m = x.mean(-1, keepdims=True)
    v = ((x - m) ** 2).mean(-1, keepdims=True)
    return (x - m) * lax.rsqrt(v + 1e-5)


def _dot(a, b):
    return jnp.dot(a, b, preferred_element_type=f32)


def _dotT(a, b):
    return lax.dot_general(a, b, (((0,), (0,)), ((), ())),
                           preferred_element_type=f32)


def _iota_r(n):
    return lax.broadcasted_iota(i32, (1, n), 1)


def _iota_c(n):
    return lax.broadcasted_iota(i32, (n, 1), 0)


def _k_means_body(emb_b, cl_b, o_meansT, acc, cnt):
    step = pl.program_id(0)

    @pl.when(step == 0)
    def _():
        acc[...] = jnp.zeros_like(acc)
        cnt[...] = jnp.zeros_like(cnt)

    hc = (cl_b[...] == _iota_r(_CP)).astype(f32)
    acc[...] += _dotT(emb_b[...], hc)
    cnt[...] += jnp.sum(hc, axis=0, keepdims=True)

    @pl.when(step == pl.num_programs(0) - 1)
    def _():
        mT = acc[...] / jnp.maximum(cnt[...], 1.0)
        nrm = jnp.sqrt(jnp.sum(mT * mT, axis=0, keepdims=True))
        o_meansT[...] = mT / (nrm + 1e-12)


def _tc_means(emb, clusters_col):
    grid = _N // _NB
    return pl.pallas_call(
        _k_means_body,
        grid=(grid,),
        in_specs=[
            pl.BlockSpec((_NB, _EMB), lambda i: (i, 0)),
            pl.BlockSpec((_NB, 1), lambda i: (i, 0)),
        ],
        out_specs=pl.BlockSpec((_EMB, _CP), lambda i: (0, 0)),
        out_shape=jax.ShapeDtypeStruct((_EMB, _CP), f32),
        scratch_shapes=[pltpu.VMEM((_EMB, _CP), f32), pltpu.VMEM((1, _CP), f32)],
    )(emb, clusters_col)


def _k_super_body(means_r, meansT_r, wb_r, o_idx, o_sew):
    m = means_r[...]
    mT = meansT_r[...]
    r2 = jnp.sum(m * m, axis=1, keepdims=True)
    c2 = jnp.sum(mT * mT, axis=0, keepdims=True)
    d2 = jnp.maximum(r2 + c2 - 2.0 * _dot(m, mT), 0.0)
    ic, ir = _iota_c(_CP), _iota_r(_CP)
    bad = (ic == ir) | (ir >= _C)
    x = -(d2 + jnp.where(bad, 1e9, 0.0))
    idxs, vals = [], []
    for _ in range(_KS):
        best = jnp.max(x, axis=1, keepdims=True)
        am = jnp.min(jnp.where(x == best, ir, _CP), axis=1, keepdims=True)
        idxs.append(am)
        vals.append(best)
        x = jnp.where(ir == am, -jnp.inf, x)
    o_idx[...] = jnp.concatenate(idxs, axis=1)
    negd = jnp.concatenate(vals, axis=1)
    w = wb_r[0, 0]
    b = wb_r[0, 1]
    o_sew[...] = jax.nn.sigmoid(negd * w + b)


def _tc_super(means, meansT, wb):
    return pl.pallas_call(
        _k_super_body,
        in_specs=[
            pl.BlockSpec((_CP, _EMB), lambda: (0, 0)),
            pl.BlockSpec((_EMB, _CP), lambda: (0, 0)),
            pl.BlockSpec((1, 2), lambda: (0, 0)),
        ],
        out_specs=[
            pl.BlockSpec((_CP, _KS), lambda: (0, 0)),
            pl.BlockSpec((_CP, _KS), lambda: (0, 0)),
        ],
        out_shape=[
            jax.ShapeDtypeStruct((_CP, _KS), i32),
            jax.ShapeDtypeStruct((_CP, _KS), f32),
        ],
    )(means, meansT, wb)


def _k_bi_body(emb_b, meansT_r, w_r, o_mw, o_dinv, accd):
    step = pl.program_id(0)

    @pl.when(step == 0)
    def _():
        accd[...] = jnp.zeros_like(accd)

    e = emb_b[...]
    mT = meansT_r[...]
    e2 = jnp.sum(e * e, axis=1, keepdims=True)
    m2 = jnp.sum(mT * mT, axis=0, keepdims=True)
    d2 = jnp.maximum(e2 + m2 - 2.0 * _dot(e, mT), 0.0)
    ir = _iota_r(_CP)
    x = -(d2 + jnp.where(ir >= _C, 1e9, 0.0))
    w = w_r[0, 0]
    mw = jnp.zeros_like(d2)
    for _ in range(_KB):
        best = jnp.max(x, axis=1, keepdims=True)
        am = jnp.min(jnp.where(x == best, ir, _CP), axis=1, keepdims=True)
        wk = jnp.exp(best * w)
        mw = mw + jnp.where(ir == am, wk, 0.0)
        x = jnp.where(ir == am, -jnp.inf, x)
    o_mw[...] = mw
    accd[...] += jnp.sum(mw, axis=0, keepdims=True)
    o_dinv[...] = 1.0 / jnp.maximum(accd[...], 1e-12)


def _tc_bi(emb, meansT, w):
    grid = _N // _NB
    return pl.pallas_call(
        _k_bi_body,
        grid=(grid,),
        in_specs=[
            pl.BlockSpec((_NB, _EMB), lambda i: (i, 0)),
            pl.BlockSpec((_EMB, _CP), lambda i: (0, 0)),
            pl.BlockSpec((1, 1), lambda i: (0, 0)),
        ],
        out_specs=[
            pl.BlockSpec((_NB, _CP), lambda i: (i, 0)),
            pl.BlockSpec((1, _CP), lambda i: (0, 0)),
        ],
        out_shape=[
            jax.ShapeDtypeStruct((_N, _CP), f32),
            jax.ShapeDtypeStruct((1, _CP), f32),
        ],
        scratch_shapes=[pltpu.VMEM((1, _CP), f32)],
    )(emb, meansT, w)


def _k_sninit_body(mw_b, nodes_b, means_r, dinvT_r, w1_r, b1_r, w2_r, b2_r,
                   w1a_r, w1b_r, o_s, o_a, o_b, acc):
    step = pl.program_id(0)

    @pl.when(step == 0)
    def _():
        acc[...] = jnp.zeros_like(acc)

    nb = nodes_b[...]
    acc[...] += _dotT(mw_b[...], nb)
    o_a[...] = _dot(nb, w1a_r[...])
    o_b[...] = _dot(nb, w1b_r[...])

    @pl.when(step == pl.num_programs(0) - 1)
    def _():
        snr = acc[...] * dinvT_r[...]
        h = _ln(jax.nn.relu(_dot(snr, w1_r[...]) + b1_r[...]))
        o = _ln(jax.nn.relu(_dot(h, w2_r[...]) + b2_r[...]))
        o_s[...] = jnp.concatenate([means_r[...], o], axis=1)


def _tc_sn_init(mw, nodes, means, dinvT, w1, b1, w2, b2, w1a, w1b):
    grid = _N // _NB
    return pl.pallas_call(
        _k_sninit_body,
        grid=(grid,),
        in_specs=[
            pl.BlockSpec((_NB, _CP), lambda i: (i, 0)),
            pl.BlockSpec((_NB, _L), lambda i: (i, 0)),
            pl.BlockSpec((_CP, _EMB), lambda i: (0, 0)),
            pl.BlockSpec((_CP, 1), lambda i: (0, 0)),
            pl.BlockSpec((_L, _L), lambda i: (0, 0)),
            pl.BlockSpec((1, _L), lambda i: (0, 0)),
            pl.BlockSpec((_L, _L - _EMB), lambda i: (0, 0)),
            pl.BlockSpec((1, _L - _EMB), lambda i: (0, 0)),
            pl.BlockSpec((_L, _L), lambda i: (0, 0)),
            pl.BlockSpec((_L, _L), lambda i: (0, 0)),
        ],
        out_specs=[
            pl.BlockSpec((_CP, _L), lambda i: (0, 0)),
            pl.BlockSpec((_NB, _L), lambda i: (i, 0)),
            pl.BlockSpec((_NB, _L), lambda i: (i, 0)),
        ],
        out_shape=[
            jax.ShapeDtypeStruct((_CP, _L), f32),
            jax.ShapeDtypeStruct((_N, _L), f32),
            jax.ShapeDtypeStruct((_N, _L), f32),
        ],
        scratch_shapes=[pltpu.VMEM((_CP, _L), f32)],
    )(mw, nodes, means, dinvT, w1, b1, w2, b2, w1a, w1b)


def _k_aggn2s_body(mw_b, nodes_b, dinvT_r, o_agg, acc):
    step = pl.program_id(0)

    @pl.when(step == 0)
    def _():
        acc[...] = jnp.zeros_like(acc)

    acc[...] += _dotT(mw_b[...], nodes_b[...])

    @pl.when(step == pl.num_programs(0) - 1)
    def _():
        o_agg[...] = acc[...] * dinvT_r[...]


def _tc_aggn2s(mw, nodes, dinvT):
    grid = _N // _NB
    return pl.pallas_call(
        _k_aggn2s_body,
        grid=(grid,),
        in_specs=[
            pl.BlockSpec((_NB, _CP), lambda i: (i, 0)),
            pl.BlockSpec((_NB, _L), lambda i: (i, 0)),
            pl.BlockSpec((_CP, 1), lambda i: (0, 0)),
        ],
        out_specs=pl.BlockSpec((_CP, _L), lambda i: (0, 0)),
        out_shape=jax.ShapeDtypeStruct((_CP, _L), f32),
        scratch_shapes=[pltpu.VMEM((_CP, _L), f32)],
    )(mw, nodes, dinvT)


def _k_seinit_body(sg0_b, sg1_b, s_r, wa_r, wb_r, b1_r, w2_r, b2_r, o_se):
    ir = _iota_r(_CP)
    s = s_r[...]
    h0 = (sg0_b[...] == ir).astype(f32)
    h1 = (sg1_b[...] == ir).astype(f32)
    h = _ln(jax.nn.relu(_dot(_dot(h0, s), wa_r[...])
                        + _dot(_dot(h1, s), wb_r[...]) + b1_r[...]))
    o_se[...] = _ln(jax.nn.relu(_dot(h, w2_r[...]) + b2_r[...]))


def _tc_se_init(sg0c, sg1c, s, wa, wb, b1, w2, b2):
    grid = _SE // _SEB
    return pl.pallas_call(
        _k_seinit_body,
        grid=(grid,),
        in_specs=[
            pl.BlockSpec((_SEB, 1), lambda i: (i, 0)),
            pl.BlockSpec((_SEB, 1), lambda i: (i, 0)),
            pl.BlockSpec((_CP, _L), lambda i: (0, 0)),
            pl.BlockSpec((_L, _L), lambda i: (0, 0)),
            pl.BlockSpec((_L, _L), lambda i: (0, 0)),
            pl.BlockSpec((1, _L), lambda i: (0, 0)),
            pl.BlockSpec((_L, _L), lambda i: (0, 0)),
            pl.BlockSpec((1, _L), lambda i: (0, 0)),
        ],
        out_specs=pl.BlockSpec((_SEB, _L), lambda i: (i, 0)),
        out_shape=jax.ShapeDtypeStruct((_SE, _L), f32),
    )(sg0c, sg1c, s, wa, wb, b1, w2, b2)


def _k_seupd_body(sg0_b, sg1_b, sg1r_b, sew_b, se_b, s_r,
                  wa_r, wb_r, wc_r, b1_r, w2_r, b2_r, o_se, o_agg):
    step = pl.program_id(0)

    @pl.when(step == 0)
    def _():
        o_agg[...] = jnp.zeros_like(o_agg)

    ir = _iota_r(_CP)
    s = s_r[...]
    se = se_b[...]
    h0 = (sg0_b[...] == ir).astype(f32)
    h1 = (sg1_b[...] == ir).astype(f32)
    h = _ln(jax.nn.relu(_dot(_dot(h0, s), wa_r[...])
                        + _dot(_dot(h1, s), wb_r[...])
                        + _dot(se, wc_r[...]) + b1_r[...]))
    se_new = se + _ln(jax.nn.relu(_dot(h, w2_r[...]) + b2_r[...]))
    o_se[...] = se_new
    h1t = (_iota_c(_CP) == sg1r_b[0]).astype(f32)
    o_agg[...] += _dot(h1t, se_new * sew_b[...])


def _tc_se_update(sg0c, sg1c, sg1r3, sewc, se, s, wa, wb, wc, b1, w2, b2):
    grid = _SE // _SEB
    return pl.pallas_call(
        _k_seupd_body,
        grid=(grid,),
        in_specs=[
            pl.BlockSpec((_SEB, 1), lambda i: (i, 0)),
            pl.BlockSpec((_SEB, 1), lambda i: (i, 0)),
            pl.BlockSpec((1, 1, _SEB), lambda i: (i, 0, 0)),
            pl.BlockSpec((_SEB, 1), lambda i: (i, 0)),
            pl.BlockSpec((_SEB, _L), lambda i: (i, 0)),
            pl.BlockSpec((_CP, _L), lambda i: (0, 0)),
            pl.BlockSpec((_L, _L), lambda i: (0, 0)),
            pl.BlockSpec((_L, _L), lambda i: (0, 0)),
            pl.BlockSpec((_L, _L), lambda i: (0, 0)),
            pl.BlockSpec((1, _L), lambda i: (0, 0)),
            pl.BlockSpec((_L, _L), lambda i: (0, 0)),
            pl.BlockSpec((1, _L), lambda i: (0, 0)),
        ],
        out_specs=[
            pl.BlockSpec((_SEB, _L), lambda i: (i, 0)),
            pl.BlockSpec((_CP, _L), lambda i: (0, 0)),
        ],
        out_shape=[
            jax.ShapeDtypeStruct((_SE, _L), f32),
            jax.ShapeDtypeStruct((_CP, _L), f32),
        ],
    )(sg0c, sg1c, sg1r3, sewc, se, s, wa, wb, wc, b1, w2, b2)


def _k_snupd_body(s_r, aggse_r, aggn2s_r, dinvT_r,
                  wa_r, wb_r, wc_r, b1_r, w2_r, b2_r, o_s, o_ssc):
    s = s_r[...]
    h = _ln(jax.nn.relu(_dot(s, wa_r[...]) + _dot(aggse_r[...], wb_r[...])
                        + _dot(aggn2s_r[...], wc_r[...]) + b1_r[...]))
    s_new = s + _ln(jax.nn.relu(_dot(h, w2_r[...]) + b2_r[...]))
    o_s[...] = s_new
    o_ssc[...] = s_new * dinvT_r[...]


def _tc_sn_update(s, aggse, aggn2s, dinvT, wa, wb, wc, b1, w2, b2):
    specs = [
        pl.BlockSpec((_CP, _L), lambda: (0, 0)),
        pl.BlockSpec((_CP, _L), lambda: (0, 0)),
        pl.BlockSpec((_CP, _L), lambda: (0, 0)),
        pl.BlockSpec((_CP, 1), lambda: (0, 0)),
        pl.BlockSpec((_L, _L), lambda: (0, 0)),
        pl.BlockSpec((_L, _L), lambda: (0, 0)),
        pl.BlockSpec((_L, _L), lambda: (0, 0)),
        pl.BlockSpec((1, _L), lambda: (0, 0)),
        pl.BlockSpec((_L, _L), lambda: (0, 0)),
        pl.BlockSpec((1, _L), lambda: (0, 0)),
    ]
    return pl.pallas_call(
        _k_snupd_body,
        in_specs=specs,
        out_specs=[
            pl.BlockSpec((_CP, _L), lambda: (0, 0)),
            pl.BlockSpec((_CP, _L), lambda: (0, 0)),
        ],
        out_shape=[
            jax.ShapeDtypeStruct((_CP, _L), f32),
            jax.ShapeDtypeStruct((_CP, _L), f32),
        ],
    )(s, aggse, aggn2s, dinvT, wa, wb, wc, b1, w2, b2)


def _k_edge_body(g_b, e_b, wc_r, b1_r, w2_r, b2_r, o_e):
    e = e_b[...]
    h = _ln(jax.nn.relu(g_b[...] + _dot(e, wc_r[...]) + b1_r[...]))
    o_e[...] = e + _ln(jax.nn.relu(_dot(h, w2_r[...]) + b2_r[...]))


def _tc_edge(g, e, wc, b1, w2, b2):
    grid = _E // _EB
    return pl.pallas_call(
        _k_edge_body,
        grid=(grid,),
        in_specs=[
            pl.BlockSpec((_EB, _L), lambda i: (i, 0)),
            pl.BlockSpec((_EB, _L), lambda i: (i, 0)),
            pl.BlockSpec((_L, _L), lambda i: (0, 0)),
            pl.BlockSpec((1, _L), lambda i: (0, 0)),
            pl.BlockSpec((_L, _L), lambda i: (0, 0)),
            pl.BlockSpec((1, _L), lambda i: (0, 0)),
        ],
        out_specs=pl.BlockSpec((_EB, _L), lambda i: (i, 0)),
        out_shape=jax.ShapeDtypeStruct((_E, _L), f32),
    )(g, e, wc, b1, w2, b2)


def _k_node_body(nodes_b, p0_b, p1_b, mw_b, ssc_r,
                 wa_r, wb_r, wc_r, b1_r, w2_r, b2_r, w1a_r, w1b_r,
                 o_n, o_a, o_b):
    n = nodes_b[...]
    agge = p0_b[...] + p1_b[...]
    aggs2n = _dot(mw_b[...], ssc_r[...])
    h = _ln(jax.nn.relu(_dot(n, wa_r[...]) + _dot(agge, wb_r[...])
                        + _dot(aggs2n, wc_r[...]) + b1_r[...]))
    n_new = n + _ln(jax.nn.relu(_dot(h, w2_r[...]) + b2_r[...]))
    o_n[...] = n_new
    o_a[...] = _dot(n_new, w1a_r[...])
    o_b[...] = _dot(n_new, w1b_r[...])


def _tc_node(nodes, p0, p1, mw, ssc, wa, wb, wc, b1, w2, b2, w1a, w1b):
    grid = _N // _NB
    return pl.pallas_call(
        _k_node_body,
        grid=(grid,),
        in_specs=[
            pl.BlockSpec((_NB, _L), lambda i: (i, 0)),
            pl.BlockSpec((_NB, _L), lambda i: (i, 0)),
            pl.BlockSpec((_NB, _L), lambda i: (i, 0)),
            pl.BlockSpec((_NB, _CP), lambda i: (i, 0)),
            pl.BlockSpec((_CP, _L), lambda i: (0, 0)),
            pl.BlockSpec((_L, _L), lambda i: (0, 0)),
            pl.BlockSpec((_L, _L), lambda i: (0, 0)),
            pl.BlockSpec((_L, _L), lambda i: (0, 0)),
            pl.BlockSpec((1, _L), lambda i: (0, 0)),
            pl.BlockSpec((_L, _L), lambda i: (0, 0)),
            pl.BlockSpec((1, _L), lambda i: (0, 0)),
            pl.BlockSpec((_L, _L), lambda i: (0, 0)),
            pl.BlockSpec((_L, _L), lambda i: (0, 0)),
        ],
        out_specs=[
            pl.BlockSpec((_NB, _L), lambda i: (i, 0)),
            pl.BlockSpec((_NB, _L), lambda i: (i, 0)),
            pl.BlockSpec((_NB, _L), lambda i: (i, 0)),
        ],
        out_shape=[
            jax.ShapeDtypeStruct((_N, _L), f32),
            jax.ShapeDtypeStruct((_N, _L), f32),
            jax.ShapeDtypeStruct((_N, _L), f32),
        ],
    )(nodes, p0, p1, mw, ssc, wa, wb, wc, b1, w2, b2, w1a, w1b)


def _k_out_body(nodes_b, w1_r, b1_r, w2_r, b2_r, o_b):
    h = _ln(jax.nn.relu(_dot(nodes_b[...], w1_r[...]) + b1_r[...]))
    o = _dot(h, w2_r[...]) + b2_r[...]
    nrm = jnp.sqrt(jnp.sum(o * o, axis=1, keepdims=True))
    o_b[...] = o / (nrm + 1e-12)


def _tc_out(nodes, w1, b1, w2, b2):
    grid = _N // _NB
    return pl.pallas_call(
        _k_out_body,
        grid=(grid,),
        in_specs=[
            pl.BlockSpec((_NB, _L), lambda i: (i, 0)),
            pl.BlockSpec((_L, _L), lambda i: (0, 0)),
            pl.BlockSpec((1, _L), lambda i: (0, 0)),
            pl.BlockSpec((_L, _EMB), lambda i: (0, 0)),
            pl.BlockSpec((1, _EMB), lambda i: (0, 0)),
        ],
        out_specs=pl.BlockSpec((_NB, _EMB), lambda i: (i, 0)),
        out_shape=jax.ShapeDtypeStruct((_N, _EMB), f32),
    )(nodes, w1, b1, w2, b2)


def _sc_gather(a, b, g0m, g1m, g0t, g1t):
    mesh = plsc.VectorSubcoreMesh(core_axis_name="c", subcore_axis_name="s")

    @functools.partial(
        pl.kernel, mesh=mesh,
        out_type=jax.ShapeDtypeStruct((_E, _L), f32),
        scratch_types=[
            pltpu.VMEM((_NCHG, _CHG), i32),
            pltpu.VMEM((_NCHG, _CHG), i32),
            pltpu.VMEM((1, _TAIL), i32),
            pltpu.VMEM((1, _TAIL), i32),
            [pltpu.VMEM((_CHG, _L), f32)] * _GBUF_G,
            [pltpu.VMEM((_CHG, _L), f32)] * _GBUF_G,
            [pltpu.SemaphoreType.DMA] * _GBUF_G,
        ],
        name="sc_gather_edges",
    )
    def k(a_h, b_h, g0_h, g1_h, g0t_h, g1t_h, out_h,
          i0, i1, t0, t1, bas, bbs, sems):
        wid = lax.axis_index("s") * 2 + lax.axis_index("c")
        pltpu.sync_copy(g0_h.at[wid], i0)
        pltpu.sync_copy(g1_h.at[wid], i1)
        pltpu.sync_copy(g0t_h.at[wid], t0)
        pltpu.sync_copy(g1t_h.at[wid], t1)

        def addbuf(ba, bb, nrows):
            def addrow(r, c2):
                for cc in range(_L // 16):
                    sl = pl.ds(cc * 16, 16)
                    ba[r, sl] = ba[r, sl] + bb[r, sl]
                return c2

            lax.fori_loop(0, nrows, addrow, 0, unroll=4)

        def group(gg, carry):
            j0 = gg * _GBUF_G
            handles = []
            for p in range(_GBUF_G):
                ha = pltpu.async_copy(a_h.at[i0.at[j0 + p]], bas[p], sems[p])
                hb = pltpu.async_copy(b_h.at[i1.at[j0 + p]], bbs[p], sems[p])
                handles.append((ha, hb))
            for p in range(_GBUF_G):
                ha, hb = handles[p]
                ha.wait()
                hb.wait()
                addbuf(bas[p], bbs[p], _CHG)
                base = wid * _EPW + (j0 + p) * _CHG
                pltpu.sync_copy(bas[p], out_h.at[pl.ds(base, _CHG)])
            return carry

        lax.fori_loop(0, _NCHG // _GBUF_G, group, 0)

        ta = bas[0].at[pl.ds(0, _TAIL)]
        tb = bbs[0].at[pl.ds(0, _TAIL)]
        pltpu.async_copy(a_h.at[t0.at[0]], ta, sems[0]).wait()
        pltpu.async_copy(b_h.at[t1.at[0]], tb, sems[0]).wait()
        addbuf(bas[0], bbs[0], _TAIL)
        pltpu.sync_copy(ta, out_h.at[pl.ds(wid * _EPW + _NCHG * _CHG, _TAIL)])

    return k(a, b, g0m, g1m, g0t, g1t)


_NPAD = 10240


def _sc_scatter(vals, g1r):
    mesh = plsc.VectorSubcoreMesh(core_axis_name="c", subcore_axis_name="s")
    rows_per_sub = _NPAD // 16

    @functools.partial(
        pl.kernel, mesh=mesh,
        out_type=jax.ShapeDtypeStruct((2, _NPAD, _L), f32),
        scratch_types=[
            pltpu.VMEM((_GBUF_S, _CHS), i32),
            [pltpu.VMEM((_CHS, _L), f32)] * _GBUF_S,
            pltpu.VMEM((80, _L), f32),
            pltpu.VMEM_SHARED((_NPAD, _L), f32),
            [pltpu.SemaphoreType.DMA] * _GBUF_S,
            pltpu.SemaphoreType.DMA,
        ],
        name="sc_scatter_edges",
    )
    def k(v_h, g1_h, out_h, idx, bufs, zbuf, acc, sems, sem_s):
        cid = lax.axis_index("c")
        sid = lax.axis_index("s")
        wid = sid * 2 + cid

        def zrow(r, c2):
            for cc in range(_L // 16):
                zbuf[r, pl.ds(cc * 16, 16)] = jnp.zeros((16,), f32)
            return c2

        lax.fori_loop(0, 80, zrow, 0)
        for t in range(rows_per_sub // 80):
            pltpu.sync_copy(zbuf, acc.at[pl.ds(sid * rows_per_sub + t * 80,
                                               80)])
        plsc.subcore_barrier()

        def group(gg, carry):
            j0 = gg * _GBUF_S
            pltpu.sync_copy(g1_h.at[wid, gg], idx)
            handles = []
            for p in range(_GBUF_S):
                base = wid * _EPW + (j0 + p) * _CHS
                handles.append(pltpu.async_copy(
                    v_h.at[pl.ds(base, _CHS)], bufs[p], sems[p]))
            sc_handles = []
            for p in range(_GBUF_S):
                handles[p].wait()
                sc_handles.append(pltpu.async_copy(
                    bufs[p], acc.at[idx.at[p]], sem_s, add=True))
            for h in sc_handles:
                h.wait()
            return carry

        lax.fori_loop(0, _NCHS // _GBUF_S, group, 0)
        plsc.subcore_barrier()
        pltpu.sync_copy(acc.at[pl.ds(sid * rows_per_sub, rows_per_sub)],
                        out_h.at[cid, pl.ds(sid * rows_per_sub, rows_per_sub)])

    return k(vals, g1r)


def kernel(x, embeddings, nodes, edges, graph, clusters, params):
    p = params
    ew1, eb1 = p["cell_edge"][0]
    ew2, eb2 = p["cell_edge"][1]
    ew1a, ew1b, ew1c = ew1[:_L], ew1[_L:2 * _L], ew1[2 * _L:]

    clc = clusters.reshape(_N, 1)
    meansT = _tc_means(embeddings, clc)
    means = meansT.T

    wb_super = jnp.stack([p["gc_super_w"], p["gc_super_b"]]).reshape(1, 2)
    idxs, sewk = _tc_super(means, meansT, wb_super)
    src = idxs[:_C].reshape(-1)
    dst = jnp.repeat(jnp.arange(_C, dtype=i32), _KS)
    sg0 = jnp.concatenate([src, dst])
    sg1 = jnp.concatenate([dst, src])
    sewh = sewk[:_C].reshape(-1)
    sew = jnp.concatenate([sewh, sewh])
    sg0c = sg0.reshape(_SE, 1)
    sg1c = sg1.reshape(_SE, 1)
    sg1r3 = sg1.reshape(_SE // _SEB, 1, _SEB)
    sewc = sew.reshape(_SE, 1)

    mw, dinv = _tc_bi(embeddings, meansT, p["gc_bi_w"].reshape(1, 1))
    dinvT = dinv.reshape(_CP, 1)

    (sw1, sb1), (sw2, sb2) = p["sn_enc"]
    s, a, b = _tc_sn_init(mw, nodes, means, dinvT, sw1, sb1.reshape(1, -1),
                          sw2, sb2.reshape(1, -1), ew1a, ew1b)

    (qw1, qb1), (qw2, qb2) = p["se_enc"]
    se = _tc_se_init(sg0c, sg1c, s, qw1[:_L], qw1[_L:], qb1.reshape(1, -1),
                     qw2, qb2.reshape(1, -1))

    gw0 = graph[0].reshape(_NW, _EPW)
    gw1 = graph[1].reshape(_NW, _EPW)
    g0m = gw0[:, :_NCHG * _CHG].reshape(_NW, _NCHG, _CHG)
    g1m = gw1[:, :_NCHG * _CHG].reshape(_NW, _NCHG, _CHG)
    g0t = gw0[:, _NCHG * _CHG:].reshape(_NW, 1, _TAIL)
    g1t = gw1[:, _NCHG * _CHG:].reshape(_NW, 1, _TAIL)
    g1rs = graph[1].reshape(_NW, _NCHS // _GBUF_S, _GBUF_S, _CHS)

    (dw1, db1), (dw2, db2) = p["cell_sedge"]
    (uw1, ub1), (uw2, ub2) = p["cell_snode"]
    (nw1, nb1), (nw2, nb2) = p["cell_node"]

    for _ in range(_ITERS):
        g = _sc_gather(a, b, g0m, g1m, g0t, g1t)
        edges = _tc_edge(g, edges, ew1c, eb1.reshape(1, -1), ew2,
                         eb2.reshape(1, -1))
        parts = _sc_scatter(edges, g1rs)[:, :_N]
        aggn2s = _tc_aggn2s(mw, nodes, dinvT)
        se, aggse = _tc_se_update(sg0c, sg1c, sg1r3, sewc, se, s,
                                  dw1[:_L], dw1[_L:2 * _L], dw1[2 * _L:],
                                  db1.reshape(1, -1), dw2, db2.reshape(1, -1))
        s, ssc = _tc_sn_update(s, aggse, aggn2s, dinvT,
                               uw1[:_L], uw1[_L:2 * _L], uw1[2 * _L:],
                               ub1.reshape(1, -1), uw2, ub2.reshape(1, -1))
        nodes, a, b = _tc_node(nodes, parts[0], parts[1], mw, ssc,
                               nw1[:_L], nw1[_L:2 * _L], nw1[2 * _L:],
                               nb1.reshape(1, -1), nw2, nb2.reshape(1, -1),
                               ew1a, ew1b)

    (ow1, ob1), (ow2, ob2) = p["out"]
    out = _tc_out(nodes, ow1, ob1.reshape(1, -1), ow2, ob2.reshape(1, -1))
    return (out, clusters)


# speedup vs baseline: 1.4349x; 1.4324x over previous
import functools

import jax
import jax.numpy as jnp
from jax import lax
from jax.experimental import pallas as pl
from jax.experimental.pallas import tpu as pltpu
from jax.experimental.pallas import tpu_sc as plsc

f32 = jnp.float32
i32 = jnp.int32

_N = 10000
_E = 320000
_L = 128
_EMB = 16
_C = 500
_CP = 512
_KS = 8
_KB = 4
_ITERS = 2
_SE = 2 * _C * _KS
_NB = 2000
_EB = 2000
_SEB = 2000
_NW = 32
_EPW = _E // _NW
_CHG = 128
_EPT = _E // 16
_NCHG = 156
_TAIL = _EPT - _NCHG * _CHG
_GBUF_G = 2
_CHS = 40
_NCHS = _EPW // _CHS
_GBUF_S = 5


def _ln(x):
    m = x.mean(-1, keepdims=True)
    v = ((x - m) ** 2).mean(-1, keepdims=True)
    return (x - m) * lax.rsqrt(v + 1e-5)


def _dot(a, b):
    return jnp.dot(a, b, preferred_element_type=f32)


def _dotT(a, b):
    return lax.dot_general(a, b, (((0,), (0,)), ((), ())),
                           preferred_element_type=f32)


def _iota_r(n):
    return lax.broadcasted_iota(i32, (1, n), 1)


def _iota_c(n):
    return lax.broadcasted_iota(i32, (n, 1), 0)


def _k_means_body(emb_b, cl_b, o_meansT, acc, cnt):
    step = pl.program_id(0)

    @pl.when(step == 0)
    def _():
        acc[...] = jnp.zeros_like(acc)
        cnt[...] = jnp.zeros_like(cnt)

    hc = (cl_b[...] == _iota_r(_CP)).astype(f32)
    acc[...] += _dotT(emb_b[...], hc)
    cnt[...] += jnp.sum(hc, axis=0, keepdims=True)

    @pl.when(step == pl.num_programs(0) - 1)
    def _():
        mT = acc[...] / jnp.maximum(cnt[...], 1.0)
        nrm = jnp.sqrt(jnp.sum(mT * mT, axis=0, keepdims=True))
        o_meansT[...] = mT / (nrm + 1e-12)


def _tc_means(emb, clusters_col):
    grid = _N // _NB
    return pl.pallas_call(
        _k_means_body,
        grid=(grid,),
        in_specs=[
            pl.BlockSpec((_NB, _EMB), lambda i: (i, 0)),
            pl.BlockSpec((_NB, 1), lambda i: (i, 0)),
        ],
        out_specs=pl.BlockSpec((_EMB, _CP), lambda i: (0, 0)),
        out_shape=jax.ShapeDtypeStruct((_EMB, _CP), f32),
        scratch_shapes=[pltpu.VMEM((_EMB, _CP), f32), pltpu.VMEM((1, _CP), f32)],
    )(emb, clusters_col)


def _k_super_body(means_r, meansT_r, wb_r, o_idx, o_sew):
    m = means_r[...]
    mT = meansT_r[...]
    r2 = jnp.sum(m * m, axis=1, keepdims=True)
    c2 = jnp.sum(mT * mT, axis=0, keepdims=True)
    d2 = jnp.maximum(r2 + c2 - 2.0 * _dot(m, mT), 0.0)
    ic, ir = _iota_c(_CP), _iota_r(_CP)
    bad = (ic == ir) | (ir >= _C)
    x = -(d2 + jnp.where(bad, 1e9, 0.0))
    idxs, vals = [], []
    for _ in range(_KS):
        best = jnp.max(x, axis=1, keepdims=True)
        am = jnp.min(jnp.where(x == best, ir, _CP), axis=1, keepdims=True)
        idxs.append(am)
        vals.append(best)
        x = jnp.where(ir == am, -jnp.inf, x)
    o_idx[...] = jnp.concatenate(idxs, axis=1)
    negd = jnp.concatenate(vals, axis=1)
    w = wb_r[0, 0]
    b = wb_r[0, 1]
    o_sew[...] = jax.nn.sigmoid(negd * w + b)


def _tc_super(means, meansT, wb):
    return pl.pallas_call(
        _k_super_body,
        in_specs=[
            pl.BlockSpec((_CP, _EMB), lambda: (0, 0)),
            pl.BlockSpec((_EMB, _CP), lambda: (0, 0)),
            pl.BlockSpec((1, 2), lambda: (0, 0)),
        ],
        out_specs=[
            pl.BlockSpec((_CP, _KS), lambda: (0, 0)),
            pl.BlockSpec((_CP, _KS), lambda: (0, 0)),
        ],
        out_shape=[
            jax.ShapeDtypeStruct((_CP, _KS), i32),
            jax.ShapeDtypeStruct((_CP, _KS), f32),
        ],
    )(means, meansT, wb)


def _k_bi_body(emb_b, meansT_r, w_r, o_mw, o_dinv, accd):
    step = pl.program_id(0)

    @pl.when(step == 0)
    def _():
        accd[...] = jnp.zeros_like(accd)

    e = emb_b[...]
    mT = meansT_r[...]
    e2 = jnp.sum(e * e, axis=1, keepdims=True)
    m2 = jnp.sum(mT * mT, axis=0, keepdims=True)
    d2 = jnp.maximum(e2 + m2 - 2.0 * _dot(e, mT), 0.0)
    ir = _iota_r(_CP)
    x = -(d2 + jnp.where(ir >= _C, 1e9, 0.0))
    w = w_r[0, 0]
    mw = jnp.zeros_like(d2)
    for _ in range(_KB):
        best = jnp.max(x, axis=1, keepdims=True)
        am = jnp.min(jnp.where(x == best, ir, _CP), axis=1, keepdims=True)
        wk = jnp.exp(best * w)
        mw = mw + jnp.where(ir == am, wk, 0.0)
        x = jnp.where(ir == am, -jnp.inf, x)
    o_mw[...] = mw
    accd[...] += jnp.sum(mw, axis=0, keepdims=True)
    o_dinv[...] = 1.0 / jnp.maximum(accd[...], 1e-12)


def _tc_bi(emb, meansT, w):
    grid = _N // _NB
    return pl.pallas_call(
        _k_bi_body,
        grid=(grid,),
        in_specs=[
            pl.BlockSpec((_NB, _EMB), lambda i: (i, 0)),
            pl.BlockSpec((_EMB, _CP), lambda i: (0, 0)),
            pl.BlockSpec((1, 1), lambda i: (0, 0)),
        ],
        out_specs=[
            pl.BlockSpec((_NB, _CP), lambda i: (i, 0)),
            pl.BlockSpec((1, _CP), lambda i: (0, 0)),
        ],
        out_shape=[
            jax.ShapeDtypeStruct((_N, _CP), f32),
            jax.ShapeDtypeStruct((1, _CP), f32),
        ],
        scratch_shapes=[pltpu.VMEM((1, _CP), f32)],
    )(emb, meansT, w)


def _k_sninit_body(mw_b, nodes_b, means_r, dinvT_r, w1_r, b1_r, w2_r, b2_r,
                   o_s, acc):
    step = pl.program_id(0)

    @pl.when(step == 0)
    def _():
        acc[...] = jnp.zeros_like(acc)

    acc[...] += _dotT(mw_b[...], nodes_b[...])

    @pl.when(step == pl.num_programs(0) - 1)
    def _():
        snr = acc[...] * dinvT_r[...]
        h = _ln(jax.nn.relu(_dot(snr, w1_r[...]) + b1_r[...]))
        o = _ln(jax.nn.relu(_dot(h, w2_r[...]) + b2_r[...]))
        o_s[...] = jnp.concatenate([means_r[...], o], axis=1)


def _tc_sn_init(mw, nodes, means, dinvT, w1, b1, w2, b2):
    grid = _N // _NB
    return pl.pallas_call(
        _k_sninit_body,
        grid=(grid,),
        in_specs=[
            pl.BlockSpec((_NB, _CP), lambda i: (i, 0)),
            pl.BlockSpec((_NB, _L), lambda i: (i, 0)),
            pl.BlockSpec((_CP, _EMB), lambda i: (0, 0)),
            pl.BlockSpec((_CP, 1), lambda i: (0, 0)),
            pl.BlockSpec((_L, _L), lambda i: (0, 0)),
            pl.BlockSpec((1, _L), lambda i: (0, 0)),
            pl.BlockSpec((_L, _L - _EMB), lambda i: (0, 0)),
            pl.BlockSpec((1, _L - _EMB), lambda i: (0, 0)),
        ],
        out_specs=pl.BlockSpec((_CP, _L), lambda i: (0, 0)),
        out_shape=jax.ShapeDtypeStruct((_CP, _L), f32),
        scratch_shapes=[pltpu.VMEM((_CP, _L), f32)],
    )(mw, nodes, means, dinvT, w1, b1, w2, b2)


def _k_ab_body(nodes_b, w1a_r, w1b_r, o_a, o_b):
    nb = nodes_b[...]
    o_a[...] = _dot(nb, w1a_r[...])
    o_b[...] = _dot(nb, w1b_r[...])


def _tc_ab(nodes, w1a, w1b):
    grid = _N // _NB
    return pl.pallas_call(
        _k_ab_body,
        grid=(grid,),
        in_specs=[
            pl.BlockSpec((_NB, _L), lambda i: (i, 0)),
            pl.BlockSpec((_L, _L), lambda i: (0, 0)),
            pl.BlockSpec((_L, _L), lambda i: (0, 0)),
        ],
        out_specs=[
            pl.BlockSpec((_NB, _L), lambda i: (i, 0)),
            pl.BlockSpec((_NB, _L), lambda i: (i, 0)),
        ],
        out_shape=[
            jax.ShapeDtypeStruct((_N, _L), f32),
            jax.ShapeDtypeStruct((_N, _L), f32),
        ],
    )(nodes, w1a, w1b)


def _k_aggn2s_body(mw_b, nodes_b, dinvT_r, o_agg, acc):
    step = pl.program_id(0)

    @pl.when(step == 0)
    def _():
        acc[...] = jnp.zeros_like(acc)

    acc[...] += _dotT(mw_b[...], nodes_b[...])

    @pl.when(step == pl.num_programs(0) - 1)
    def _():
        o_agg[...] = acc[...] * dinvT_r[...]


def _tc_aggn2s(mw, nodes, dinvT):
    grid = _N // _NB
    return pl.pallas_call(
        _k_aggn2s_body,
        grid=(grid,),
        in_specs=[
            pl.BlockSpec((_NB, _CP), lambda i: (i, 0)),
            pl.BlockSpec((_NB, _L), lambda i: (i, 0)),
            pl.BlockSpec((_CP, 1), lambda i: (0, 0)),
        ],
        out_specs=pl.BlockSpec((_CP, _L), lambda i: (0, 0)),
        out_shape=jax.ShapeDtypeStruct((_CP, _L), f32),
        scratch_shapes=[pltpu.VMEM((_CP, _L), f32)],
    )(mw, nodes, dinvT)


def _k_seinit_body(sg0_b, sg1_b, s_r, wa_r, wb_r, b1_r, w2_r, b2_r, o_se):
    ir = _iota_r(_CP)
    s = s_r[...]
    h0 = (sg0_b[...] == ir).astype(f32)
    h1 = (sg1_b[...] == ir).astype(f32)
    h = _ln(jax.nn.relu(_dot(_dot(h0, s), wa_r[...])
                        + _dot(_dot(h1, s), wb_r[...]) + b1_r[...]))
    o_se[...] = _ln(jax.nn.relu(_dot(h, w2_r[...]) + b2_r[...]))


def _tc_se_init(sg0c, sg1c, s, wa, wb, b1, w2, b2):
    grid = _SE // _SEB
    return pl.pallas_call(
        _k_seinit_body,
        grid=(grid,),
        in_specs=[
            pl.BlockSpec((_SEB, 1), lambda i: (i, 0)),
            pl.BlockSpec((_SEB, 1), lambda i: (i, 0)),
            pl.BlockSpec((_CP, _L), lambda i: (0, 0)),
            pl.BlockSpec((_L, _L), lambda i: (0, 0)),
            pl.BlockSpec((_L, _L), lambda i: (0, 0)),
            pl.BlockSpec((1, _L), lambda i: (0, 0)),
            pl.BlockSpec((_L, _L), lambda i: (0, 0)),
            pl.BlockSpec((1, _L), lambda i: (0, 0)),
        ],
        out_specs=pl.BlockSpec((_SEB, _L), lambda i: (i, 0)),
        out_shape=jax.ShapeDtypeStruct((_SE, _L), f32),
    )(sg0c, sg1c, s, wa, wb, b1, w2, b2)


def _k_seupd_body(sg0_b, sg1_b, sg1r_b, sew_b, se_b, s_r,
                  wa_r, wb_r, wc_r, b1_r, w2_r, b2_r, o_se, o_agg):
    step = pl.program_id(0)

    @pl.when(step == 0)
    def _():
        o_agg[...] = jnp.zeros_like(o_agg)

    ir = _iota_r(_CP)
    s = s_r[...]
    se = se_b[...]
    h0 = (sg0_b[...] == ir).astype(f32)
    h1 = (sg1_b[...] == ir).astype(f32)
    h = _ln(jax.nn.relu(_dot(_dot(h0, s), wa_r[...])
                        + _dot(_dot(h1, s), wb_r[...])
                        + _dot(se, wc_r[...]) + b1_r[...]))
    se_new = se + _ln(jax.nn.relu(_dot(h, w2_r[...]) + b2_r[...]))
    o_se[...] = se_new
    h1t = (_iota_c(_CP) == sg1r_b[0]).astype(f32)
    o_agg[...] += _dot(h1t, se_new * sew_b[...])


def _tc_se_update(sg0c, sg1c, sg1r3, sewc, se, s, wa, wb, wc, b1, w2, b2):
    grid = _SE // _SEB
    return pl.pallas_call(
        _k_seupd_body,
        grid=(grid,),
        in_specs=[
            pl.BlockSpec((_SEB, 1), lambda i: (i, 0)),
            pl.BlockSpec((_SEB, 1), lambda i: (i, 0)),
            pl.BlockSpec((1, 1, _SEB), lambda i: (i, 0, 0)),
            pl.BlockSpec((_SEB, 1), lambda i: (i, 0)),
            pl.BlockSpec((_SEB, _L), lambda i: (i, 0)),
            pl.BlockSpec((_CP, _L), lambda i: (0, 0)),
            pl.BlockSpec((_L, _L), lambda i: (0, 0)),
            pl.BlockSpec((_L, _L), lambda i: (0, 0)),
            pl.BlockSpec((_L, _L), lambda i: (0, 0)),
            pl.BlockSpec((1, _L), lambda i: (0, 0)),
            pl.BlockSpec((_L, _L), lambda i: (0, 0)),
            pl.BlockSpec((1, _L), lambda i: (0, 0)),
        ],
        out_specs=[
            pl.BlockSpec((_SEB, _L), lambda i: (i, 0)),
            pl.BlockSpec((_CP, _L), lambda i: (0, 0)),
        ],
        out_shape=[
            jax.ShapeDtypeStruct((_SE, _L), f32),
            jax.ShapeDtypeStruct((_CP, _L), f32),
        ],
    )(sg0c, sg1c, sg1r3, sewc, se, s, wa, wb, wc, b1, w2, b2)


def _k_snupd_body(s_r, aggse_r, aggn2s_r, dinvT_r,
                  wa_r, wb_r, wc_r, b1_r, w2_r, b2_r, o_s, o_ssc):
    s = s_r[...]
    h = _ln(jax.nn.relu(_dot(s, wa_r[...]) + _dot(aggse_r[...], wb_r[...])
                        + _dot(aggn2s_r[...], wc_r[...]) + b1_r[...]))
    s_new = s + _ln(jax.nn.relu(_dot(h, w2_r[...]) + b2_r[...]))
    o_s[...] = s_new
    o_ssc[...] = s_new * dinvT_r[...]


def _tc_sn_update(s, aggse, aggn2s, dinvT, wa, wb, wc, b1, w2, b2):
    specs = [
        pl.BlockSpec((_CP, _L), lambda: (0, 0)),
        pl.BlockSpec((_CP, _L), lambda: (0, 0)),
        pl.BlockSpec((_CP, _L), lambda: (0, 0)),
        pl.BlockSpec((_CP, 1), lambda: (0, 0)),
        pl.BlockSpec((_L, _L), lambda: (0, 0)),
        pl.BlockSpec((_L, _L), lambda: (0, 0)),
        pl.BlockSpec((_L, _L), lambda: (0, 0)),
        pl.BlockSpec((1, _L), lambda: (0, 0)),
        pl.BlockSpec((_L, _L), lambda: (0, 0)),
        pl.BlockSpec((1, _L), lambda: (0, 0)),
    ]
    return pl.pallas_call(
        _k_snupd_body,
        in_specs=specs,
        out_specs=[
            pl.BlockSpec((_CP, _L), lambda: (0, 0)),
            pl.BlockSpec((_CP, _L), lambda: (0, 0)),
        ],
        out_shape=[
            jax.ShapeDtypeStruct((_CP, _L), f32),
            jax.ShapeDtypeStruct((_CP, _L), f32),
        ],
    )(s, aggse, aggn2s, dinvT, wa, wb, wc, b1, w2, b2)


def _k_edge_body(ga_b, gb_b, e_b, wc_r, b1_r, w2_r, b2_r, o_e):
    e = e_b[...]
    g = ga_b[0] + gb_b[0]
    h = _ln(jax.nn.relu(g + _dot(e, wc_r[...]) + b1_r[...]))
    o_e[...] = e + _ln(jax.nn.relu(_dot(h, w2_r[...]) + b2_r[...]))


def _tc_edge(g2, e, wc, b1, w2, b2, blk0):
    grid = (_E // 2) // _EB
    return pl.pallas_call(
        _k_edge_body,
        grid=(grid,),
        in_specs=[
            pl.BlockSpec((1, _EB, _L), lambda i: (0, i + blk0, 0)),
            pl.BlockSpec((1, _EB, _L), lambda i: (1, i + blk0, 0)),
            pl.BlockSpec((_EB, _L), lambda i: (i, 0)),
            pl.BlockSpec((_L, _L), lambda i: (0, 0)),
            pl.BlockSpec((1, _L), lambda i: (0, 0)),
            pl.BlockSpec((_L, _L), lambda i: (0, 0)),
            pl.BlockSpec((1, _L), lambda i: (0, 0)),
        ],
        out_specs=pl.BlockSpec((_EB, _L), lambda i: (i, 0)),
        out_shape=jax.ShapeDtypeStruct((_E // 2, _L), f32),
    )(g2, g2, e, wc, b1, w2, b2)


def _k_node_body(nodes_b, pa0_b, pa1_b, pb0_b, pb1_b, mw_b, ssc_r,
                 wa_r, wb_r, wc_r, b1_r, w2_r, b2_r, w1a_r, w1b_r,
                 o_n, o_a, o_b):
    n = nodes_b[...]
    agge = pa0_b[0] + pa1_b[0] + pb0_b[0] + pb1_b[0]
    aggs2n = _dot(mw_b[...], ssc_r[...])
    h = _ln(jax.nn.relu(_dot(n, wa_r[...]) + _dot(agge, wb_r[...])
                        + _dot(aggs2n, wc_r[...]) + b1_r[...]))
    n_new = n + _ln(jax.nn.relu(_dot(h, w2_r[...]) + b2_r[...]))
    o_n[...] = n_new
    o_a[...] = _dot(n_new, w1a_r[...])
    o_b[...] = _dot(n_new, w1b_r[...])


def _tc_node(nodes, pa, pb, mw, ssc, wa, wb, wc, b1, w2, b2, w1a, w1b):
    grid = _N // _NB
    return pl.pallas_call(
        _k_node_body,
        grid=(grid,),
        in_specs=[
            pl.BlockSpec((_NB, _L), lambda i: (i, 0)),
            pl.BlockSpec((1, _NB, _L), lambda i: (0, i, 0)),
            pl.BlockSpec((1, _NB, _L), lambda i: (1, i, 0)),
            pl.BlockSpec((1, _NB, _L), lambda i: (0, i, 0)),
            pl.BlockSpec((1, _NB, _L), lambda i: (1, i, 0)),
            pl.BlockSpec((_NB, _CP), lambda i: (i, 0)),
            pl.BlockSpec((_CP, _L), lambda i: (0, 0)),
            pl.BlockSpec((_L, _L), lambda i: (0, 0)),
            pl.BlockSpec((_L, _L), lambda i: (0, 0)),
            pl.BlockSpec((_L, _L), lambda i: (0, 0)),
            pl.BlockSpec((1, _L), lambda i: (0, 0)),
            pl.BlockSpec((_L, _L), lambda i: (0, 0)),
            pl.BlockSpec((1, _L), lambda i: (0, 0)),
            pl.BlockSpec((_L, _L), lambda i: (0, 0)),
            pl.BlockSpec((_L, _L), lambda i: (0, 0)),
        ],
        out_specs=[
            pl.BlockSpec((_NB, _L), lambda i: (i, 0)),
            pl.BlockSpec((_NB, _L), lambda i: (i, 0)),
            pl.BlockSpec((_NB, _L), lambda i: (i, 0)),
        ],
        out_shape=[
            jax.ShapeDtypeStruct((_N, _L), f32),
            jax.ShapeDtypeStruct((_N, _L), f32),
            jax.ShapeDtypeStruct((_N, _L), f32),
        ],
    )(nodes, pa, pa, pb, pb, mw, ssc, wa, wb, wc, b1, w2, b2, w1a, w1b)


def _k_out_body(nodes_b, w1_r, b1_r, w2_r, b2_r, o_b):
    h = _ln(jax.nn.relu(_dot(nodes_b[...], w1_r[...]) + b1_r[...]))
    o = _dot(h, w2_r[...]) + b2_r[...]
    nrm = jnp.sqrt(jnp.sum(o * o, axis=1, keepdims=True))
    o_b[...] = o / (nrm + 1e-12)


def _tc_out(nodes, w1, b1, w2, b2):
    grid = _N // _NB
    return pl.pallas_call(
        _k_out_body,
        grid=(grid,),
        in_specs=[
            pl.BlockSpec((_NB, _L), lambda i: (i, 0)),
            pl.BlockSpec((_L, _L), lambda i: (0, 0)),
            pl.BlockSpec((1, _L), lambda i: (0, 0)),
            pl.BlockSpec((_L, _EMB), lambda i: (0, 0)),
            pl.BlockSpec((1, _EMB), lambda i: (0, 0)),
        ],
        out_specs=pl.BlockSpec((_NB, _EMB), lambda i: (i, 0)),
        out_shape=jax.ShapeDtypeStruct((_N, _EMB), f32),
    )(nodes, w1, b1, w2, b2)


_TPAD = 10240


def _sc_gather(a, b, gm, gt):
    nch = gm.shape[2]
    tail = gt.shape[3]
    ept = nch * _CHG + tail
    e_out = 16 * ept
    mesh = plsc.VectorSubcoreMesh(core_axis_name="c", subcore_axis_name="s")

    @functools.partial(
        pl.kernel, mesh=mesh,
        out_type=jax.ShapeDtypeStruct((2, e_out, _L), f32),
        scratch_types=[
            pltpu.VMEM((_GBUF_G, _CHG), i32),
            pltpu.VMEM((1, tail), i32),
            [pltpu.VMEM((_CHG, _L), f32)] * _GBUF_G,
            pltpu.VMEM_SHARED((_TPAD, _L), f32),
            [pltpu.SemaphoreType.DMA] * _GBUF_G,
        ],
        name="sc_gather_edges",
    )
    def k(a_h, b_h, gm_h, gt_h, out_h, ig, it, bufs, tab, sems):
        cid = lax.axis_index("c")
        sid = lax.axis_index("s")
        slab = _TPAD // 16

        @pl.when(cid == 0)
        def _():
            @pl.when(sid < 15)
            def _():
                pltpu.sync_copy(a_h.at[pl.ds(sid * slab, slab)],
                                tab.at[pl.ds(sid * slab, slab)])

            @pl.when(sid == 15)
            def _():
                pltpu.sync_copy(a_h.at[pl.ds(15 * slab, _N - 15 * slab)],
                                tab.at[pl.ds(15 * slab, _N - 15 * slab)])

        @pl.when(cid == 1)
        def _():
            @pl.when(sid < 15)
            def _():
                pltpu.sync_copy(b_h.at[pl.ds(sid * slab, slab)],
                                tab.at[pl.ds(sid * slab, slab)])

            @pl.when(sid == 15)
            def _():
                pltpu.sync_copy(b_h.at[pl.ds(15 * slab, _N - 15 * slab)],
                                tab.at[pl.ds(15 * slab, _N - 15 * slab)])

        plsc.subcore_barrier()

        def group(gg, carry):
            j0 = gg * _GBUF_G
            pltpu.sync_copy(gm_h.at[cid, sid, pl.ds(j0, _GBUF_G)], ig)
            handles = []
            for p in range(_GBUF_G):
                handles.append(
                    pltpu.async_copy(tab.at[ig.at[p]], bufs[p], sems[p]))
            for p in range(_GBUF_G):
                handles[p].wait()
                base = sid * ept + (j0 + p) * _CHG
                pltpu.sync_copy(bufs[p], out_h.at[cid, pl.ds(base, _CHG)])
            return carry

        lax.fori_loop(0, nch // _GBUF_G, group, 0)

        pltpu.sync_copy(gt_h.at[cid, sid], it)
        tbuf = bufs[0].at[pl.ds(0, tail)]
        pltpu.async_copy(tab.at[it.at[0]], tbuf, sems[0]).wait()
        pltpu.sync_copy(tbuf, out_h.at[cid, pl.ds(sid * ept + nch * _CHG,
                                                  tail)])

    return k(a, b, gm, gt)


_NPAD = 10240


def _sc_scatter(vals, g1r):
    ngr = g1r.shape[1]
    epw = ngr * _GBUF_S * _CHS
    mesh = plsc.VectorSubcoreMesh(core_axis_name="c", subcore_axis_name="s")
    rows_per_sub = _NPAD // 16

    @functools.partial(
        pl.kernel, mesh=mesh,
        out_type=jax.ShapeDtypeStruct((2, _NPAD, _L), f32),
        scratch_types=[
            pltpu.VMEM((_GBUF_S, _CHS), i32),
            [pltpu.VMEM((_CHS, _L), f32)] * _GBUF_S,
            pltpu.VMEM((80, _L), f32),
            pltpu.VMEM_SHARED((_NPAD, _L), f32),
            [pltpu.SemaphoreType.DMA] * _GBUF_S,
            pltpu.SemaphoreType.DMA,
        ],
        name="sc_scatter_edges",
    )
    def k(v_h, g1_h, out_h, idx, bufs, zbuf, acc, sems, sem_s):
        cid = lax.axis_index("c")
        sid = lax.axis_index("s")
        wid = sid * 2 + cid

        def zrow(r, c2):
            for cc in range(_L // 16):
                zbuf[r, pl.ds(cc * 16, 16)] = jnp.zeros((16,), f32)
            return c2

        lax.fori_loop(0, 80, zrow, 0)
        for t in range(rows_per_sub // 80):
            pltpu.sync_copy(zbuf, acc.at[pl.ds(sid * rows_per_sub + t * 80,
                                               80)])
        plsc.subcore_barrier()

        def group(gg, carry):
            j0 = gg * _GBUF_S
            pltpu.sync_copy(g1_h.at[wid, gg], idx)
            handles = []
            for p in range(_GBUF_S):
                base = wid * epw + (j0 + p) * _CHS
                handles.append(pltpu.async_copy(
                    v_h.at[pl.ds(base, _CHS)], bufs[p], sems[p]))
            sc_handles = []
            for p in range(_GBUF_S):
                handles[p].wait()
                sc_handles.append(pltpu.async_copy(
                    bufs[p], acc.at[idx.at[p]], sem_s, add=True))
            for h in sc_handles:
                h.wait()
            return carry

        lax.fori_loop(0, ngr, group, 0)
        plsc.subcore_barrier()
        pltpu.sync_copy(acc.at[pl.ds(sid * rows_per_sub, rows_per_sub)],
                        out_h.at[cid, pl.ds(sid * rows_per_sub, rows_per_sub)])

    return k(vals, g1r)


def kernel(x, embeddings, nodes, edges, graph, clusters, params):
    p = params
    ew1, eb1 = p["cell_edge"][0]
    ew2, eb2 = p["cell_edge"][1]
    ew1a, ew1b, ew1c = ew1[:_L], ew1[_L:2 * _L], ew1[2 * _L:]

    eh = _E // 2
    gw = jnp.stack([graph[0], graph[1]]).reshape(2, 16, _EPT)
    gm = gw[:, :, :_NCHG * _CHG].reshape(2, 16, _NCHG, _CHG)
    gt = gw[:, :, _NCHG * _CHG:].reshape(2, 16, 1, _TAIL)
    g1rss = [graph[1, h * eh:(h + 1) * eh].reshape(
        _NW, eh // _NW // (_GBUF_S * _CHS), _GBUF_S, _CHS) for h in range(2)]
    e0, e1 = edges[:eh], edges[eh:]

    a, b = _tc_ab(nodes, ew1a, ew1b)
    g = _sc_gather(a, b, gm, gt)

    clc = clusters.reshape(_N, 1)
    meansT = _tc_means(embeddings, clc)
    means = meansT.T

    wb_super = jnp.stack([p["gc_super_w"], p["gc_super_b"]]).reshape(1, 2)
    idxs, sewk = _tc_super(means, meansT, wb_super)
    src = idxs[:_C].reshape(-1)
    dst = jnp.repeat(jnp.arange(_C, dtype=i32), _KS)
    sg0 = jnp.concatenate([src, dst])
    sg1 = jnp.concatenate([dst, src])
    sewh = sewk[:_C].reshape(-1)
    sew = jnp.concatenate([sewh, sewh])
    sg0c = sg0.reshape(_SE, 1)
    sg1c = sg1.reshape(_SE, 1)
    sg1r3 = sg1.reshape(_SE // _SEB, 1, _SEB)
    sewc = sew.reshape(_SE, 1)

    mw, dinv = _tc_bi(embeddings, meansT, p["gc_bi_w"].reshape(1, 1))
    dinvT = dinv.reshape(_CP, 1)

    (sw1, sb1), (sw2, sb2) = p["sn_enc"]
    s = _tc_sn_init(mw, nodes, means, dinvT, sw1, sb1.reshape(1, -1),
                    sw2, sb2.reshape(1, -1))

    (qw1, qb1), (qw2, qb2) = p["se_enc"]
    se = _tc_se_init(sg0c, sg1c, s, qw1[:_L], qw1[_L:], qb1.reshape(1, -1),
                     qw2, qb2.reshape(1, -1))

    (dw1, db1), (dw2, db2) = p["cell_sedge"]
    (uw1, ub1), (uw2, ub2) = p["cell_snode"]
    (nw1, nb1), (nw2, nb2) = p["cell_node"]

    for it in range(_ITERS):
        if it > 0:
            g = _sc_gather(a, b, gm, gt)
        aggn2s = _tc_aggn2s(mw, nodes, dinvT)
        se, aggse = _tc_se_update(sg0c, sg1c, sg1r3, sewc, se, s,
                                  dw1[:_L], dw1[_L:2 * _L], dw1[2 * _L:],
                                  db1.reshape(1, -1), dw2, db2.reshape(1, -1))
        s, ssc = _tc_sn_update(s, aggse, aggn2s, dinvT,
                               uw1[:_L], uw1[_L:2 * _L], uw1[2 * _L:],
                               ub1.reshape(1, -1), uw2, ub2.reshape(1, -1))
        e0 = _tc_edge(g, e0, ew1c, eb1.reshape(1, -1), ew2,
                      eb2.reshape(1, -1), 0)
        pa = _sc_scatter(e0, g1rss[0])
        e1 = _tc_edge(g, e1, ew1c, eb1.reshape(1, -1), ew2,
                      eb2.reshape(1, -1), eh // _EB)
        pb = _sc_scatter(e1, g1rss[1])
        nodes, a, b = _tc_node(nodes, pa, pb, mw, ssc,
                               nw1[:_L], nw1[_L:2 * _L], nw1[2 * _L:],
                               nb1.reshape(1, -1), nw2, nb2.reshape(1, -1),
                               ew1a, ew1b)

    (ow1, ob1), (ow2, ob2) = p["out"]
    out = _tc_out(nodes, ow1, ob1.reshape(1, -1), ow2, ob2.reshape(1, -1))
    return (out, clusters)
